# baseline (speedup 1.0000x reference)
import jax
import jax.numpy as jnp
from jax import lax
from jax.experimental import pallas as pl
from jax.experimental.pallas import tpu as pltpu
from jax.experimental.pallas import tpu_sc as plsc

N = 10000
R = 16
D = 256
H = 128
E = 160000
NS = 16
EPT = E // NS
CK = 80
NCH = EPT // CK
NPAD = 10240
RPT = NPAD // NS
BRWS = 128
BN = 1000
NB = N // BN


def _lin_body(x_ref, w_ref, b_ref, o_ref):
    o_ref[...] = jnp.dot(x_ref[...].astype(jnp.bfloat16),
                         w_ref[...].astype(jnp.bfloat16),
                         preferred_element_type=jnp.float32) + b_ref[...]


def _lin(x, w, b2d):
    return pl.pallas_call(
        _lin_body,
        grid=(NB,),
        in_specs=[
            pl.BlockSpec((BN, D), lambda i: (i, 0)),
            pl.BlockSpec((D, D), lambda i: (0, 0)),
            pl.BlockSpec((1, D), lambda i: (0, 0)),
        ],
        out_specs=pl.BlockSpec((BN, D), lambda i: (i, 0)),
        out_shape=jax.ShapeDtypeStruct((N, D), jnp.float32),
    )(x, w, b2d)


def _relmm_body(x_ref, w_ref, oa_ref, ob_ref):
    res = jnp.dot(x_ref[...].astype(jnp.bfloat16),
                  w_ref[0].astype(jnp.bfloat16),
                  preferred_element_type=jnp.float32)
    oa_ref[...] = res[:, :H]
    ob_ref[...] = res[:, H:]


def _relmm(x, w):
    return pl.pallas_call(
        _relmm_body,
        grid=(NB, R),
        in_specs=[
            pl.BlockSpec((BN, D), lambda i, r: (i, 0)),
            pl.BlockSpec((1, D, D), lambda i, r: (r, 0, 0)),
        ],
        out_specs=[
            pl.BlockSpec((BN, H), lambda i, r: (r * NB + i, 0)),
            pl.BlockSpec((BN, H), lambda i, r: (r * NB + i, 0)),
        ],
        out_shape=[
            jax.ShapeDtypeStruct((R * N, H), jnp.float32),
            jax.ShapeDtypeStruct((R * N, H), jnp.float32),
        ],
    )(x, w)


def _combine_body(x_ref, w_ref, b_ref, aa_ref, ab_ref, o_ref):
    res = jnp.dot(x_ref[...].astype(jnp.bfloat16),
                  w_ref[...].astype(jnp.bfloat16),
                  preferred_element_type=jnp.float32)
    agg = jnp.concatenate([aa_ref[0], ab_ref[0]], axis=1)
    o_ref[...] = jnp.maximum(res + b_ref[...] + agg, 0.0)


def _combine(x, root, b2d, acc):
    return pl.pallas_call(
        _combine_body,
        grid=(NB,),
        in_specs=[
            pl.BlockSpec((BN, D), lambda i: (i, 0)),
            pl.BlockSpec((D, D), lambda i: (0, 0)),
            pl.BlockSpec((1, D), lambda i: (0, 0)),
            pl.BlockSpec((1, BN, H), lambda i: (0, i, 0)),
            pl.BlockSpec((1, BN, H), lambda i: (1, i, 0)),
        ],
        out_specs=pl.BlockSpec((BN, D), lambda i: (i, 0)),
        out_shape=jax.ShapeDtypeStruct((N, D), jnp.float32),
    )(x, root, b2d, acc, acc)


_MESH = plsc.VectorSubcoreMesh(core_axis_name="c", subcore_axis_name="s")
_SC_PARAMS = pltpu.CompilerParams(needs_layout_passes=False)


def _prep_body(src_hbm, dst_hbm, et_hbm, gidx_hbm, scale_hbm,
               src_v, dst_v, et_v, gidx_v, keys_v, scale_v,
               keyb_v, ones_v, cnt_v, cnt_sp, sem):
    cid = lax.axis_index("c")
    sid = lax.axis_index("s")
    base = sid * EPT

    def zfill(i, _):
        scale_v[pl.ds(i * 16, 16)] = lax.broadcast(jnp.float32(0.0), (16,))
        return _

    lax.fori_loop(0, EPT // 16, zfill, None)
    pltpu.sync_copy(scale_v, cnt_sp.at[pl.ds(base, EPT)])

    pltpu.sync_copy(src_hbm.at[pl.ds(base, EPT)], src_v)
    pltpu.sync_copy(dst_hbm.at[pl.ds(base, EPT)], dst_v)
    pltpu.sync_copy(et_hbm.at[pl.ds(base, EPT)], et_v)
    plsc.subcore_barrier()

    for v in range(CK // 16):
        ones_v[pl.ds(v * 16, 16)] = lax.broadcast(jnp.float32(1.0), (16,))

    def count_chunk(ch, _):
        o = ch * CK
        for v in range(CK // 16):
            sl = pl.ds(v * 16, 16)
            osl = pl.ds(o + v * 16, 16)
            et16 = et_v[osl]
            keyb_v[sl] = et16 * N + dst_v[osl]
            gidx_v[osl] = et16 * N + src_v[osl]
            keys_v[osl] = keyb_v[sl]
        pltpu.sync_copy(ones_v, cnt_sp.at[keyb_v], add=True)
        return _

    lax.fori_loop(0, NCH, count_chunk, None)
    plsc.subcore_barrier()

    def scale_chunk(ch, _):
        o = ch * CK
        pltpu.async_copy(cnt_sp.at[keys_v.at[pl.ds(o, CK)]], cnt_v, sem).wait()
        for v in range(CK // 16):
            c16 = cnt_v[pl.ds(v * 16, 16)]
            scale_v[pl.ds(o + v * 16, 16)] = 1.0 / jnp.maximum(c16, 1.0)
        return _

    lax.fori_loop(0, NCH, scale_chunk, None)

    @pl.when(cid == 0)
    def _():
        pltpu.sync_copy(gidx_v, gidx_hbm.at[pl.ds(base, EPT)])
        pltpu.sync_copy(scale_v, scale_hbm.at[pl.ds(base, EPT)])


def _prep(src, dst, et):
    return pl.kernel(
        _prep_body,
        out_type=[
            jax.ShapeDtypeStruct((E,), jnp.int32),
            jax.ShapeDtypeStruct((E,), jnp.float32),
        ],
        mesh=_MESH,
        compiler_params=_SC_PARAMS,
        scratch_types=[
            pltpu.VMEM((EPT,), jnp.int32),
            pltpu.VMEM((EPT,), jnp.int32),
            pltpu.VMEM((EPT,), jnp.int32),
            pltpu.VMEM((EPT,), jnp.int32),
            pltpu.VMEM((EPT,), jnp.int32),
            pltpu.VMEM((EPT,), jnp.float32),
            pltpu.VMEM((CK,), jnp.int32),
            pltpu.VMEM((CK,), jnp.float32),
            pltpu.VMEM((CK,), jnp.float32),
            pltpu.VMEM_SHARED((E,), jnp.float32),
            pltpu.SemaphoreType.DMA,
        ],
    )(src, dst, et)


def _agg_body(xra_hbm, xrb_hbm, gidx_hbm, dst_hbm, scale_hbm, acc_hbm,
              gb0, gb1, gb2, di0, di1, di2, sb0, sb1, sb2, dc0, dc1, dc2,
              rw0, rw1, rw2, bounce_v,
              acc_sp, lsem0, lsem1, lsem2, gsem0, gsem1, gsem2,
              ssem0, ssem1, ssem2):
    cid = lax.axis_index("c")
    sid = lax.axis_index("s")
    base = sid * EPT
    gb = (gb0, gb1, gb2)
    din = (di0, di1, di2)
    sb = (sb0, sb1, sb2)
    dsc = (dc0, dc1, dc2)
    rows = (rw0, rw1, rw2)
    lsem = (lsem0, lsem1, lsem2)
    gsem = (gsem0, gsem1, gsem2)
    ssem = (ssem0, ssem1, ssem2)

    def zfill(i, _):
        bounce_v[i, pl.ds(0, 16)] = lax.broadcast(jnp.float32(0.0), (16,))
        for v in range(1, H // 16):
            bounce_v[i, pl.ds(v * 16, 16)] = bounce_v[i, pl.ds(0, 16)]
        return _

    lax.fori_loop(0, BRWS, zfill, None)
    for k in range(RPT // BRWS):
        pltpu.sync_copy(bounce_v,
                        acc_sp.at[pl.ds(sid * RPT + k * BRWS, BRWS)])
    plsc.subcore_barrier()

    def edge_loop(xr_hbm):
        def fire_lin(k_off, s):
            o = base + k_off * CK
            pltpu.async_copy(gidx_hbm.at[pl.ds(o, CK)], gb[s], lsem[s])
            pltpu.async_copy(dst_hbm.at[pl.ds(o, CK)], din[s], lsem[s])
            pltpu.async_copy(scale_hbm.at[pl.ds(o, CK)], sb[s], lsem[s])

        def wait_lin(s):
            pltpu.make_async_copy(gidx_hbm.at[pl.ds(base, CK)], gb[s],
                                  lsem[s]).wait()
            pltpu.make_async_copy(dst_hbm.at[pl.ds(base, CK)], din[s],
                                  lsem[s]).wait()
            pltpu.make_async_copy(scale_hbm.at[pl.ds(base, CK)], sb[s],
                                  lsem[s]).wait()

        def fire_gather(s):
            pltpu.async_copy(xr_hbm.at[gb[s]], rows[s], gsem[s])

        def wait_gather(s):
            pltpu.make_async_copy(xr_hbm.at[gb[s]], rows[s], gsem[s]).wait()

        def fire_scatter(s):
            pltpu.async_copy(rows[s], acc_sp.at[dsc[s]], ssem[s], add=True)

        def wait_scatter(s):
            pltpu.make_async_copy(rows[s], acc_sp.at[dsc[s]], ssem[s]).wait()

        def scale_rows(s):
            def edge(j, _):
                for u in range(4):
                    bc = plsc.load_gather(
                        sb[s], [lax.broadcast(j * 4 + u, (16,))])
                    for v in range(H // 16):
                        sl = pl.ds(v * 16, 16)
                        rows[s][j * 4 + u, sl] = rows[s][j * 4 + u, sl] * bc
                return _

            lax.fori_loop(0, CK // 4, edge, None)

        def copy_din(s):
            for v in range(CK // 16):
                dsc[s][pl.ds(v * 16, 16)] = din[s][pl.ds(v * 16, 16)]

        def step(k, s, s1, s2, skip_sc_wait=False, fire_next_lin=True,
                 knext=None, klin=None):
            wait_gather(s)
            copy_din(s)
            wait_lin(s1)
            if not skip_sc_wait:
                wait_scatter(s1)
            fire_gather(s1)
            scale_rows(s)
            fire_scatter(s)
            if fire_next_lin:
                fire_lin(klin, s2)

        fire_lin(0, 0)
        fire_lin(1, 1)
        wait_lin(0)
        fire_gather(0)
        step(0, 0, 1, 2, skip_sc_wait=True, klin=2)
        step(1, 1, 2, 0, skip_sc_wait=True, klin=3)

        def triple(i, _):
            k = 3 * i + 2
            step(k, 2, 0, 1, klin=jnp.minimum(k + 2, NCH - 1))
            step(k + 1, 0, 1, 2, klin=jnp.minimum(k + 3, NCH - 1))
            step(k + 2, 1, 2, 0, klin=jnp.minimum(k + 4, NCH - 1))
            return _

        lax.fori_loop(0, (NCH - 2) // 3, triple, None)

        wait_gather((NCH) % 3)
        wait_lin((NCH + 1) % 3)
        wait_scatter((NCH - 2) % 3)
        wait_scatter((NCH - 1) % 3)

    @pl.when(cid == 0)
    def _():
        edge_loop(xra_hbm)

    @pl.when(cid == 1)
    def _():
        edge_loop(xrb_hbm)

    plsc.subcore_barrier()
    for k in range(RPT // BRWS):
        pltpu.sync_copy(acc_sp.at[pl.ds(sid * RPT + k * BRWS, BRWS)],
                        bounce_v)
        pltpu.sync_copy(bounce_v,
                        acc_hbm.at[cid, pl.ds(sid * RPT + k * BRWS, BRWS)])


def _agg(xra, xrb, gidx, dst, scale):
    return pl.kernel(
        _agg_body,
        out_type=jax.ShapeDtypeStruct((2, NPAD, H), jnp.float32),
        mesh=_MESH,
        compiler_params=_SC_PARAMS,
        scratch_types=(
            [pltpu.VMEM((CK,), jnp.int32) for _ in range(3)]
            + [pltpu.VMEM((CK,), jnp.int32) for _ in range(3)]
            + [pltpu.VMEM((CK,), jnp.float32) for _ in range(3)]
            + [pltpu.VMEM((CK,), jnp.int32) for _ in range(3)]
            + [pltpu.VMEM((CK, H), jnp.float32) for _ in range(3)]
            + [pltpu.VMEM((BRWS, H), jnp.float32)]
            + [pltpu.VMEM_SHARED((NPAD, H), jnp.float32)]
            + [pltpu.SemaphoreType.DMA for _ in range(9)]
        ),
    )(xra, xrb, gidx, dst, scale)


def kernel(node_index, edge_index, edge_type, node_emb, lin_W, lin_b,
           W1, root1, b1, W2, root2, b2):
    src = edge_index[0].astype(jnp.int32)
    dst = edge_index[1].astype(jnp.int32)
    et = edge_type.astype(jnp.int32)

    x = _lin(node_emb, lin_W.T, lin_b.reshape(1, D))
    gidx, scale = _prep(src, dst, et)

    for (W, root, b) in ((W1, root1, b1), (W2, root2, b2)):
        xra, xrb = _relmm(x, W)
        acc = _agg(xra, xrb, gidx, dst, scale)
        x = _combine(x, root, b.reshape(1, D), acc)
    return x


# speedup vs baseline: 1.3117x; 1.3117x over previous
import jax
import jax.numpy as jnp
from jax import lax
from jax.experimental import pallas as pl
from jax.experimental.pallas import tpu as pltpu
from jax.experimental.pallas import tpu_sc as plsc

N = 10000
R = 16
D = 256
H = 128
E = 160000
NS = 16
EPT = E // NS
CK = 80
NCH = EPT // CK
NPAD = 10240
RPT = NPAD // NS
BRWS = 128
BN = 400
NB = N // BN


def _head_body(e_ref, lw_ref, lb_ref, w_ref, h_ref, oa_ref, ob_ref):
    x = jnp.dot(e_ref[...].astype(jnp.bfloat16),
                lw_ref[...].astype(jnp.bfloat16),
                preferred_element_type=jnp.float32) + lb_ref[...]
    h_ref[...] = x
    xb = x.astype(jnp.bfloat16)
    for r in range(R):
        res = jnp.dot(xb, w_ref[r].astype(jnp.bfloat16),
                      preferred_element_type=jnp.float32)
        oa_ref[pl.ds(r * BN, BN), :] = res[:, :H]
        ob_ref[pl.ds(r * BN, BN), :] = res[:, H:]


def _head(emb, lw, lb2d, w):
    return pl.pallas_call(
        _head_body,
        grid=(NB,),
        in_specs=[
            pl.BlockSpec((BN, D), lambda i: (i, 0)),
            pl.BlockSpec((D, D), lambda i: (0, 0)),
            pl.BlockSpec((1, D), lambda i: (0, 0)),
            pl.BlockSpec((R, D, D), lambda i: (0, 0, 0)),
        ],
        out_specs=[
            pl.BlockSpec((BN, D), lambda i: (i, 0)),
            pl.BlockSpec((R * BN, H), lambda i: (i, 0)),
            pl.BlockSpec((R * BN, H), lambda i: (i, 0)),
        ],
        out_shape=[
            jax.ShapeDtypeStruct((N, D), jnp.float32),
            jax.ShapeDtypeStruct((R * N, H), jnp.float32),
            jax.ShapeDtypeStruct((R * N, H), jnp.float32),
        ],
    )(emb, lw, lb2d, w)


def _mid_body(x_ref, rt_ref, b_ref, aa_ref, ab_ref, w_ref,
              h_ref, oa_ref, ob_ref):
    res = jnp.dot(x_ref[...].astype(jnp.bfloat16),
                  rt_ref[...].astype(jnp.bfloat16),
                  preferred_element_type=jnp.float32)
    agg = jnp.concatenate([aa_ref[0], ab_ref[0]], axis=1)
    h = jnp.maximum(res + b_ref[...] + agg, 0.0)
    h_ref[...] = h
    hb = h.astype(jnp.bfloat16)
    for r in range(R):
        res = jnp.dot(hb, w_ref[r].astype(jnp.bfloat16),
                      preferred_element_type=jnp.float32)
        oa_ref[pl.ds(r * BN, BN), :] = res[:, :H]
        ob_ref[pl.ds(r * BN, BN), :] = res[:, H:]


def _mid(x, root, b2d, acc, w):
    return pl.pallas_call(
        _mid_body,
        grid=(NB,),
        in_specs=[
            pl.BlockSpec((BN, D), lambda i: (i, 0)),
            pl.BlockSpec((D, D), lambda i: (0, 0)),
            pl.BlockSpec((1, D), lambda i: (0, 0)),
            pl.BlockSpec((1, BN, H), lambda i: (0, i, 0)),
            pl.BlockSpec((1, BN, H), lambda i: (1, i, 0)),
            pl.BlockSpec((R, D, D), lambda i: (0, 0, 0)),
        ],
        out_specs=[
            pl.BlockSpec((BN, D), lambda i: (i, 0)),
            pl.BlockSpec((R * BN, H), lambda i: (i, 0)),
            pl.BlockSpec((R * BN, H), lambda i: (i, 0)),
        ],
        out_shape=[
            jax.ShapeDtypeStruct((N, D), jnp.float32),
            jax.ShapeDtypeStruct((R * N, H), jnp.float32),
            jax.ShapeDtypeStruct((R * N, H), jnp.float32),
        ],
    )(x, root, b2d, acc, acc, w)


def _combine_body(x_ref, w_ref, b_ref, aa_ref, ab_ref, o_ref):
    res = jnp.dot(x_ref[...].astype(jnp.bfloat16),
                  w_ref[...].astype(jnp.bfloat16),
                  preferred_element_type=jnp.float32)
    agg = jnp.concatenate([aa_ref[0], ab_ref[0]], axis=1)
    o_ref[...] = jnp.maximum(res + b_ref[...] + agg, 0.0)


def _combine(x, root, b2d, acc):
    return pl.pallas_call(
        _combine_body,
        grid=(NB,),
        in_specs=[
            pl.BlockSpec((BN, D), lambda i: (i, 0)),
            pl.BlockSpec((D, D), lambda i: (0, 0)),
            pl.BlockSpec((1, D), lambda i: (0, 0)),
            pl.BlockSpec((1, BN, H), lambda i: (0, i, 0)),
            pl.BlockSpec((1, BN, H), lambda i: (1, i, 0)),
        ],
        out_specs=pl.BlockSpec((BN, D), lambda i: (i, 0)),
        out_shape=jax.ShapeDtypeStruct((N, D), jnp.float32),
    )(x, root, b2d, acc, acc)


_MESH = plsc.VectorSubcoreMesh(core_axis_name="c", subcore_axis_name="s")
_SC_PARAMS = pltpu.CompilerParams(needs_layout_passes=False)


def _prep_body(src_hbm, dst_hbm, et_hbm, gidx_hbm, scale_hbm,
               src_v, dst_v, et_v, gidx_v, keys_v, scale_v,
               keyb_v, ones_v, cnt_v, cnt_sp, sem):
    cid = lax.axis_index("c")
    sid = lax.axis_index("s")
    base = sid * EPT

    def zfill(i, _):
        scale_v[pl.ds(i * 16, 16)] = lax.broadcast(jnp.float32(0.0), (16,))
        return _

    lax.fori_loop(0, EPT // 16, zfill, None)
    pltpu.sync_copy(scale_v, cnt_sp.at[pl.ds(base, EPT)])

    pltpu.sync_copy(src_hbm.at[pl.ds(base, EPT)], src_v)
    pltpu.sync_copy(dst_hbm.at[pl.ds(base, EPT)], dst_v)
    pltpu.sync_copy(et_hbm.at[pl.ds(base, EPT)], et_v)
    plsc.subcore_barrier()

    for v in range(CK // 16):
        ones_v[pl.ds(v * 16, 16)] = lax.broadcast(jnp.float32(1.0), (16,))

    def count_chunk(ch, _):
        o = ch * CK
        for v in range(CK // 16):
            sl = pl.ds(v * 16, 16)
            osl = pl.ds(o + v * 16, 16)
            et16 = et_v[osl]
            keyb_v[sl] = et16 * N + dst_v[osl]
            src16 = src_v[osl]
            blk16 = src16 // BN
            gidx_v[osl] = blk16 * (R * BN) + et16 * BN + (src16 - blk16 * BN)
            keys_v[osl] = keyb_v[sl]
        pltpu.sync_copy(ones_v, cnt_sp.at[keyb_v], add=True)
        return _

    lax.fori_loop(0, NCH, count_chunk, None)
    plsc.subcore_barrier()

    def scale_chunk(ch, _):
        o = ch * CK
        pltpu.async_copy(cnt_sp.at[keys_v.at[pl.ds(o, CK)]], cnt_v, sem).wait()
        for v in range(CK // 16):
            c16 = cnt_v[pl.ds(v * 16, 16)]
            scale_v[pl.ds(o + v * 16, 16)] = 1.0 / jnp.maximum(c16, 1.0)
        return _

    lax.fori_loop(0, NCH, scale_chunk, None)

    @pl.when(cid == 0)
    def _():
        pltpu.sync_copy(gidx_v, gidx_hbm.at[pl.ds(base, EPT)])
        pltpu.sync_copy(scale_v, scale_hbm.at[pl.ds(base, EPT)])


def _prep(src, dst, et):
    return pl.kernel(
        _prep_body,
        out_type=[
            jax.ShapeDtypeStruct((E,), jnp.int32),
            jax.ShapeDtypeStruct((E,), jnp.float32),
        ],
        mesh=_MESH,
        compiler_params=_SC_PARAMS,
        scratch_types=[
            pltpu.VMEM((EPT,), jnp.int32),
            pltpu.VMEM((EPT,), jnp.int32),
            pltpu.VMEM((EPT,), jnp.int32),
            pltpu.VMEM((EPT,), jnp.int32),
            pltpu.VMEM((EPT,), jnp.int32),
            pltpu.VMEM((EPT,), jnp.float32),
            pltpu.VMEM((CK,), jnp.int32),
            pltpu.VMEM((CK,), jnp.float32),
            pltpu.VMEM((CK,), jnp.float32),
            pltpu.VMEM_SHARED((E,), jnp.float32),
            pltpu.SemaphoreType.DMA,
        ],
    )(src, dst, et)


def _agg_body(xra_hbm, xrb_hbm, gidx_hbm, dst_hbm, scale_hbm, acc_hbm,
              gb0, gb1, gb2, di0, di1, di2, sb0, sb1, sb2, dc0, dc1, dc2,
              rw0, rw1, rw2, bounce_v,
              acc_sp, lsem0, lsem1, lsem2, gsem0, gsem1, gsem2,
              ssem0, ssem1, ssem2):
    cid = lax.axis_index("c")
    sid = lax.axis_index("s")
    base = sid * EPT
    gb = (gb0, gb1, gb2)
    din = (di0, di1, di2)
    sb = (sb0, sb1, sb2)
    dsc = (dc0, dc1, dc2)
    rows = (rw0, rw1, rw2)
    lsem = (lsem0, lsem1, lsem2)
    gsem = (gsem0, gsem1, gsem2)
    ssem = (ssem0, ssem1, ssem2)

    def zfill(i, _):
        bounce_v[i, pl.ds(0, 16)] = lax.broadcast(jnp.float32(0.0), (16,))
        for v in range(1, H // 16):
            bounce_v[i, pl.ds(v * 16, 16)] = bounce_v[i, pl.ds(0, 16)]
        return _

    lax.fori_loop(0, BRWS, zfill, None)
    for k in range(RPT // BRWS):
        pltpu.sync_copy(bounce_v,
                        acc_sp.at[pl.ds(sid * RPT + k * BRWS, BRWS)])
    plsc.subcore_barrier()

    def edge_loop(xr_hbm):
        def fire_lin(k_off, s):
            o = base + k_off * CK
            pltpu.async_copy(gidx_hbm.at[pl.ds(o, CK)], gb[s], lsem[s])
            pltpu.async_copy(dst_hbm.at[pl.ds(o, CK)], din[s], lsem[s])
            pltpu.async_copy(scale_hbm.at[pl.ds(o, CK)], sb[s], lsem[s])

        def wait_lin(s):
            pltpu.make_async_copy(gidx_hbm.at[pl.ds(base, CK)], gb[s],
                                  lsem[s]).wait()
            pltpu.make_async_copy(dst_hbm.at[pl.ds(base, CK)], din[s],
                                  lsem[s]).wait()
            pltpu.make_async_copy(scale_hbm.at[pl.ds(base, CK)], sb[s],
                                  lsem[s]).wait()

        def fire_gather(s):
            pltpu.async_copy(xr_hbm.at[gb[s]], rows[s], gsem[s])

        def wait_gather(s):
            pltpu.make_async_copy(xr_hbm.at[gb[s]], rows[s], gsem[s]).wait()

        def fire_scatter(s):
            pltpu.async_copy(rows[s], acc_sp.at[dsc[s]], ssem[s], add=True)

        def wait_scatter(s):
            pltpu.make_async_copy(rows[s], acc_sp.at[dsc[s]], ssem[s]).wait()

        def scale_rows(s):
            def edge(j, _):
                for u in range(4):
                    bc = plsc.load_gather(
                        sb[s], [lax.broadcast(j * 4 + u, (16,))])
                    for v in range(H // 16):
                        sl = pl.ds(v * 16, 16)
                        rows[s][j * 4 + u, sl] = rows[s][j * 4 + u, sl] * bc
                return _

            lax.fori_loop(0, CK // 4, edge, None)

        def copy_din(s):
            for v in range(CK // 16):
                dsc[s][pl.ds(v * 16, 16)] = din[s][pl.ds(v * 16, 16)]

        def step(k, s, s1, s2, skip_sc_wait=False, fire_next_lin=True,
                 knext=None, klin=None):
            wait_gather(s)
            copy_din(s)
            wait_lin(s1)
            if not skip_sc_wait:
                wait_scatter(s1)
            fire_gather(s1)
            scale_rows(s)
            fire_scatter(s)
            if fire_next_lin:
                fire_lin(klin, s2)

        fire_lin(0, 0)
        fire_lin(1, 1)
        wait_lin(0)
        fire_gather(0)
        step(0, 0, 1, 2, skip_sc_wait=True, klin=2)
        step(1, 1, 2, 0, skip_sc_wait=True, klin=3)

        def triple(i, _):
            k = 3 * i + 2
            step(k, 2, 0, 1, klin=jnp.minimum(k + 2, NCH - 1))
            step(k + 1, 0, 1, 2, klin=jnp.minimum(k + 3, NCH - 1))
            step(k + 2, 1, 2, 0, klin=jnp.minimum(k + 4, NCH - 1))
            return _

        lax.fori_loop(0, (NCH - 2) // 3, triple, None)

        wait_gather((NCH) % 3)
        wait_lin((NCH + 1) % 3)
        wait_scatter((NCH - 2) % 3)
        wait_scatter((NCH - 1) % 3)

    @pl.when(cid == 0)
    def _():
        edge_loop(xra_hbm)

    @pl.when(cid == 1)
    def _():
        edge_loop(xrb_hbm)

    plsc.subcore_barrier()
    for k in range(RPT // BRWS):
        pltpu.sync_copy(acc_sp.at[pl.ds(sid * RPT + k * BRWS, BRWS)],
                        bounce_v)
        pltpu.sync_copy(bounce_v,
                        acc_hbm.at[cid, pl.ds(sid * RPT + k * BRWS, BRWS)])


def _agg(xra, xrb, gidx, dst, scale):
    return pl.kernel(
        _agg_body,
        out_type=jax.ShapeDtypeStruct((2, NPAD, H), jnp.float32),
        mesh=_MESH,
        compiler_params=_SC_PARAMS,
        scratch_types=(
            [pltpu.VMEM((CK,), jnp.int32) for _ in range(3)]
            + [pltpu.VMEM((CK,), jnp.int32) for _ in range(3)]
            + [pltpu.VMEM((CK,), jnp.float32) for _ in range(3)]
            + [pltpu.VMEM((CK,), jnp.int32) for _ in range(3)]
            + [pltpu.VMEM((CK, H), jnp.float32) for _ in range(3)]
            + [pltpu.VMEM((BRWS, H), jnp.float32)]
            + [pltpu.VMEM_SHARED((NPAD, H), jnp.float32)]
            + [pltpu.SemaphoreType.DMA for _ in range(9)]
        ),
    )(xra, xrb, gidx, dst, scale)


def kernel(node_index, edge_index, edge_type, node_emb, lin_W, lin_b,
           W1, root1, b1, W2, root2, b2):
    src = edge_index[0].astype(jnp.int32)
    dst = edge_index[1].astype(jnp.int32)
    et = edge_type.astype(jnp.int32)

    gidx, scale = _prep(src, dst, et)
    x, xra, xrb = _head(node_emb, lin_W.T, lin_b.reshape(1, D), W1)
    acc = _agg(xra, xrb, gidx, dst, scale)
    h, xra2, xrb2 = _mid(x, root1, b1.reshape(1, D), acc, W2)
    acc2 = _agg(xra2, xrb2, gidx, dst, scale)
    return _combine(h, root2, b2.reshape(1, D), acc2)
